# Initial kernel scaffold; baseline (speedup 1.0000x reference)
import functools

import jax
import jax.numpy as jnp
from jax import lax
from jax.experimental import pallas as pl
from jax.experimental.pallas import tpu as pltpu
from jax.experimental.pallas import tpu_sc as plsc

N = 10000
E = 320000
D = 128

NC = 2
NS = 16
NW = NC * NS

NPAD = 10240
RPT = NPAD // NS
EW = E // NW
CH = 80
NCH = EW // CH

_mesh = plsc.VectorSubcoreMesh(core_axis_name="c", subcore_axis_name="s")


def _worker_id():
    return lax.axis_index("s") * NC + lax.axis_index("c")


@functools.partial(
    pl.kernel,
    mesh=_mesh,
    out_type=[
        jax.ShapeDtypeStruct((NC * NPAD,), jnp.float32),
        jax.ShapeDtypeStruct((NC * NPAD,), jnp.float32),
    ],
    scratch_types=[
        pltpu.VMEM((NCH, CH), jnp.int32),
        pltpu.VMEM((NCH, CH), jnp.int32),
        pltpu.VMEM((CH,), jnp.float32),
        pltpu.VMEM_SHARED((NPAD,), jnp.float32),
        pltpu.VMEM_SHARED((NPAD,), jnp.float32),
    ],
)
def _hist_sc(src_hbm, dst_hbm, zeros_hbm, outs_hbm, outd_hbm,
             src_v, dst_v, ones_v, hs_sh, hd_sh):
    c = lax.axis_index("c")
    s = lax.axis_index("s")
    w = _worker_id()

    @pl.when(s == 0)
    def _():
        pltpu.sync_copy(zeros_hbm, hs_sh)
        pltpu.sync_copy(zeros_hbm, hd_sh)

    for j in range(CH // 16):
        ones_v[pl.ds(j * 16, 16)] = jnp.ones((16,), jnp.float32)
    pltpu.sync_copy(src_hbm.at[pl.ds(w * NCH, NCH)], src_v)
    pltpu.sync_copy(dst_hbm.at[pl.ds(w * NCH, NCH)], dst_v)
    plsc.subcore_barrier()

    def body(j, carry):
        pltpu.sync_copy(ones_v, hs_sh.at[src_v.at[j]], add=True)
        pltpu.sync_copy(ones_v, hd_sh.at[dst_v.at[j]], add=True)
        return carry

    lax.fori_loop(0, NCH, body, 0)
    plsc.subcore_barrier()
    off = c * NPAD + s * RPT
    pltpu.sync_copy(hs_sh.at[pl.ds(s * RPT, RPT)], outs_hbm.at[pl.ds(off, RPT)])
    pltpu.sync_copy(hd_sh.at[pl.ds(s * RPT, RPT)], outd_hbm.at[pl.ds(off, RPT)])


@functools.partial(
    pl.kernel,
    mesh=_mesh,
    out_type=jax.ShapeDtypeStruct((NC * NPAD, D), jnp.float32),
    scratch_types=[
        pltpu.VMEM((NCH, CH), jnp.int32),
        pltpu.VMEM((NCH, CH), jnp.int32),
        pltpu.VMEM((CH, D), jnp.float32),
        pltpu.VMEM((64, D), jnp.float32),
        pltpu.VMEM_SHARED((NPAD, D), jnp.float32),
        pltpu.SemaphoreType.DMA,
    ],
)
def _agg_sc(g_hbm, src_hbm, dst_hbm, out_hbm,
            src_v, dst_v, rows_v, zv, agg_sh, sem):
    c = lax.axis_index("c")
    s = lax.axis_index("s")
    w = _worker_id()

    def zbody(i, carry):
        zv[i // 8, pl.ds((i % 8) * 16, 16)] = jnp.zeros((16,), jnp.float32)
        return carry

    lax.fori_loop(0, 64 * 8, zbody, 0)

    def zcopy(k, carry):
        pltpu.sync_copy(zv, agg_sh.at[pl.ds(s * RPT + k * 64, 64)])
        return carry

    lax.fori_loop(0, RPT // 64, zcopy, 0)

    pltpu.sync_copy(src_hbm.at[pl.ds(w * NCH, NCH)], src_v)
    pltpu.sync_copy(dst_hbm.at[pl.ds(w * NCH, NCH)], dst_v)
    plsc.subcore_barrier()

    def body(j, carry):
        pltpu.async_copy(g_hbm.at[src_v.at[j]], rows_v, sem).wait()
        pltpu.sync_copy(rows_v, agg_sh.at[dst_v.at[j]], add=True)
        return carry

    lax.fori_loop(0, NCH, body, 0)
    plsc.subcore_barrier()
    off = c * NPAD + s * RPT
    pltpu.sync_copy(agg_sh.at[pl.ds(s * RPT, RPT)], out_hbm.at[pl.ds(off, RPT)])


def _norm_tc_body(feat_ref, deg_ref, g_ref):
    d = deg_ref[:N, 0:1] + deg_ref[:N, 1:2]
    norm = lax.rsqrt(jnp.maximum(d, 1.0))
    g_ref[...] = feat_ref[...] * norm


def _norm_tc(feat, deg_t):
    return pl.pallas_call(
        _norm_tc_body,
        out_shape=jax.ShapeDtypeStruct((N, D), jnp.float32),
    )(feat, deg_t)


def _tail_tc_body(aggp_ref, deg_ref, w_ref, b_ref, gamma_ref, beta_ref,
                  a1_ref, a2_ref, out_ref):
    agg = aggp_ref[:N, :] + aggp_ref[NPAD:NPAD + N, :]
    d = deg_ref[:N, 0:1] + deg_ref[:N, 1:2]
    norm = lax.rsqrt(jnp.maximum(d, 1.0))
    h = agg * norm
    h = lax.dot_general(h, w_ref[...], (((1,), (0,)), ((), ())),
                        preferred_element_type=jnp.float32,
                        precision=lax.Precision.HIGHEST)
    h = h + b_ref[...]
    h = jnp.where(h >= 0, h, a1_ref[...] * h)
    mean = jnp.mean(h, axis=0, keepdims=True)
    var = jnp.mean((h - mean) ** 2, axis=0, keepdims=True)
    h = (h - mean) * lax.rsqrt(var + 1e-5) * gamma_ref[...] + beta_ref[...]
    out_ref[...] = jnp.where(h >= 0, h, a2_ref[...] * h)


def _tail_tc(aggp, deg_in_t, W, b, gamma, beta, a1, a2):
    return pl.pallas_call(
        _tail_tc_body,
        out_shape=jax.ShapeDtypeStruct((N, D), jnp.float32),
    )(aggp, deg_in_t, W, b.reshape(1, D), gamma.reshape(1, D),
      beta.reshape(1, D), a1.reshape(1, 1), a2.reshape(1, 1))


def kernel(feat, edge_index, W, b, a1, gamma, beta, a2):
    src = edge_index[0].astype(jnp.int32).reshape(E // CH, CH)
    dst = edge_index[1].astype(jnp.int32).reshape(E // CH, CH)
    zeros_h = jnp.zeros((NPAD,), jnp.float32)

    hs, hd = _hist_sc(src, dst, zeros_h)
    deg_out_t = hs.reshape(NC, NPAD).T
    deg_in_t = hd.reshape(NC, NPAD).T

    g = _norm_tc(feat, deg_out_t)
    aggp = _agg_sc(g, src, dst)
    return _tail_tc(aggp, deg_in_t, W, b, gamma, beta,
                    jnp.asarray(a1, jnp.float32), jnp.asarray(a2, jnp.float32))


# speedup vs baseline: 8.2578x; 8.2578x over previous
import functools

import jax
import jax.numpy as jnp
from jax import lax
from jax.experimental import pallas as pl
from jax.experimental.pallas import tpu as pltpu
from jax.experimental.pallas import tpu_sc as plsc

N = 10000
E = 320000
D = 128

NC = 2
NS = 16
NW = NC * NS

NPAD = 10240
RPT = NPAD // NS
EW = E // NW
CH = 80
NCH = EW // CH

_mesh = plsc.VectorSubcoreMesh(core_axis_name="c", subcore_axis_name="s")


def _worker_id():
    return lax.axis_index("s") * NC + lax.axis_index("c")


@functools.partial(
    pl.kernel,
    mesh=_mesh,
    out_type=[
        jax.ShapeDtypeStruct((NC * NPAD,), jnp.float32),
        jax.ShapeDtypeStruct((NC * NPAD,), jnp.float32),
    ],
    scratch_types=[
        pltpu.VMEM((NCH, CH), jnp.int32),
        pltpu.VMEM((NCH, CH), jnp.int32),
        pltpu.VMEM((CH,), jnp.float32),
        pltpu.VMEM_SHARED((NPAD,), jnp.float32),
        pltpu.VMEM_SHARED((NPAD,), jnp.float32),
    ],
)
def _hist_sc(src_hbm, dst_hbm, zeros_hbm, outs_hbm, outd_hbm,
             src_v, dst_v, ones_v, hs_sh, hd_sh):
    c = lax.axis_index("c")
    s = lax.axis_index("s")
    w = _worker_id()

    @pl.when(s == 0)
    def _():
        pltpu.sync_copy(zeros_hbm, hs_sh)
        pltpu.sync_copy(zeros_hbm, hd_sh)

    for j in range(CH // 16):
        ones_v[pl.ds(j * 16, 16)] = jnp.ones((16,), jnp.float32)
    pltpu.sync_copy(src_hbm.at[w], src_v)
    pltpu.sync_copy(dst_hbm.at[w], dst_v)
    plsc.subcore_barrier()

    def body(j, carry):
        pltpu.sync_copy(ones_v, hs_sh.at[src_v.at[j]], add=True)
        pltpu.sync_copy(ones_v, hd_sh.at[dst_v.at[j]], add=True)
        return carry

    lax.fori_loop(0, NCH, body, 0)
    plsc.subcore_barrier()
    off = c * NPAD + s * RPT
    pltpu.sync_copy(hs_sh.at[pl.ds(s * RPT, RPT)], outs_hbm.at[pl.ds(off, RPT)])
    pltpu.sync_copy(hd_sh.at[pl.ds(s * RPT, RPT)], outd_hbm.at[pl.ds(off, RPT)])


@functools.partial(
    pl.kernel,
    mesh=_mesh,
    compiler_params=pltpu.CompilerParams(use_tc_tiling_on_sc=False),
    out_type=jax.ShapeDtypeStruct((NC * N, D), jnp.float32),
    scratch_types=[
        pltpu.VMEM((NCH, CH), jnp.int32),
        pltpu.VMEM((NCH, CH), jnp.int32),
        pltpu.VMEM((CH, D), jnp.float32),
        pltpu.VMEM((64, D), jnp.float32),
        pltpu.VMEM_SHARED((10000, D), jnp.float32),
    ],
)
def _agg_sc(g_hbm, src_hbm, dst_hbm, out_hbm,
            src_v, dst_v, rows_v, zv, agg_sh):
    c = lax.axis_index("c")
    s = lax.axis_index("s")
    w = _worker_id()

    def zbody(i, carry):
        zv[i // 8, pl.ds((i % 8) * 16, 16)] = jnp.zeros((16,), jnp.float32)
        return carry

    lax.fori_loop(0, 64 * 8, zbody, 0)

    def zcopy(k, carry):
        pltpu.sync_copy(zv, agg_sh.at[pl.ds(s * 625 + k * 64, 64)])
        return carry

    lax.fori_loop(0, 9, zcopy, 0)
    pltpu.sync_copy(zv.at[pl.ds(0, 49)], agg_sh.at[pl.ds(s * 625 + 576, 49)])

    pltpu.sync_copy(src_hbm.at[w], src_v)
    pltpu.sync_copy(dst_hbm.at[w], dst_v)
    plsc.subcore_barrier()

    def body(j, carry):
        pltpu.sync_copy(g_hbm.at[src_v.at[j]], rows_v)
        pltpu.sync_copy(rows_v, agg_sh.at[dst_v.at[j]], add=True)
        return carry

    lax.fori_loop(0, NCH, body, 0)
    plsc.subcore_barrier()
    off = c * N + s * 625
    pltpu.sync_copy(agg_sh.at[pl.ds(s * 625, 625)], out_hbm.at[pl.ds(off, 625)])


def _norm_tc_body(feat_ref, deg_ref, g_ref):
    d = deg_ref[:N, 0:1] + deg_ref[:N, 1:2]
    norm = lax.rsqrt(jnp.maximum(d, 1.0))
    g_ref[...] = feat_ref[...] * norm


def _norm_tc(feat, deg_t):
    return pl.pallas_call(
        _norm_tc_body,
        out_shape=jax.ShapeDtypeStruct((N, D), jnp.float32),
    )(feat, deg_t)


def _tail_tc_body(aggp_ref, deg_ref, w_ref, b_ref, gamma_ref, beta_ref,
                  a1_ref, a2_ref, out_ref):
    agg = aggp_ref[:N, :] + aggp_ref[N:2 * N, :]
    d = deg_ref[:N, 0:1] + deg_ref[:N, 1:2]
    norm = lax.rsqrt(jnp.maximum(d, 1.0))
    h = agg * norm
    h = lax.dot_general(h, w_ref[...], (((1,), (0,)), ((), ())),
                        preferred_element_type=jnp.float32,
                        precision=lax.Precision.HIGHEST)
    h = h + b_ref[...]
    h = jnp.where(h >= 0, h, a1_ref[...] * h)
    mean = jnp.mean(h, axis=0, keepdims=True)
    var = jnp.mean((h - mean) ** 2, axis=0, keepdims=True)
    h = (h - mean) * lax.rsqrt(var + 1e-5) * gamma_ref[...] + beta_ref[...]
    out_ref[...] = jnp.where(h >= 0, h, a2_ref[...] * h)


def _tail_tc(aggp, deg_in_t, W, b, gamma, beta, a1, a2):
    return pl.pallas_call(
        _tail_tc_body,
        out_shape=jax.ShapeDtypeStruct((N, D), jnp.float32),
    )(aggp, deg_in_t, W, b.reshape(1, D), gamma.reshape(1, D),
      beta.reshape(1, D), a1.reshape(1, 1), a2.reshape(1, 1))


def kernel(feat, edge_index, W, b, a1, gamma, beta, a2):
    src = edge_index[0].astype(jnp.int32).reshape(NW, NCH, CH)
    dst = edge_index[1].astype(jnp.int32).reshape(NW, NCH, CH)
    zeros_h = jnp.zeros((NPAD,), jnp.float32)

    hs, hd = _hist_sc(src, dst, zeros_h)
    deg_out_t = hs.reshape(NC, NPAD).T
    deg_in_t = hd.reshape(NC, NPAD).T

    g = _norm_tc(feat, deg_out_t)
    aggp = _agg_sc(g, src, dst)
    return _tail_tc(aggp, deg_in_t, W, b, gamma, beta,
                    jnp.asarray(a1, jnp.float32), jnp.asarray(a2, jnp.float32))


# speedup vs baseline: 10.0438x; 1.2163x over previous
import functools

import jax
import jax.numpy as jnp
from jax import lax
from jax.experimental import pallas as pl
from jax.experimental.pallas import tpu as pltpu
from jax.experimental.pallas import tpu_sc as plsc

N = 10000
E = 320000
D = 128

NC = 2
NS = 16
NW = NC * NS

NPAD = 10240
RPT = NPAD // NS
EW = E // NW
CH = 80
NCH = EW // CH

_mesh = plsc.VectorSubcoreMesh(core_axis_name="c", subcore_axis_name="s")


def _worker_id():
    return lax.axis_index("s") * NC + lax.axis_index("c")


@functools.partial(
    pl.kernel,
    mesh=_mesh,
    out_type=[
        jax.ShapeDtypeStruct((NC * NPAD,), jnp.float32),
        jax.ShapeDtypeStruct((NC * NPAD,), jnp.float32),
    ],
    scratch_types=[
        pltpu.VMEM((NCH, CH), jnp.int32),
        pltpu.VMEM((NCH, CH), jnp.int32),
        pltpu.VMEM((CH,), jnp.float32),
        pltpu.VMEM_SHARED((NPAD,), jnp.float32),
        pltpu.VMEM_SHARED((NPAD,), jnp.float32),
    ],
)
def _hist_sc(src_hbm, dst_hbm, zeros_hbm, outs_hbm, outd_hbm,
             src_v, dst_v, ones_v, hs_sh, hd_sh):
    c = lax.axis_index("c")
    s = lax.axis_index("s")
    w = _worker_id()

    @pl.when(s == 0)
    def _():
        pltpu.sync_copy(zeros_hbm, hs_sh)
        pltpu.sync_copy(zeros_hbm, hd_sh)

    for j in range(CH // 16):
        ones_v[pl.ds(j * 16, 16)] = jnp.ones((16,), jnp.float32)
    pltpu.sync_copy(src_hbm.at[w], src_v)
    pltpu.sync_copy(dst_hbm.at[w], dst_v)
    plsc.subcore_barrier()

    def body(j, carry):
        pltpu.sync_copy(ones_v, hs_sh.at[src_v.at[j]], add=True)
        pltpu.sync_copy(ones_v, hd_sh.at[dst_v.at[j]], add=True)
        return carry

    lax.fori_loop(0, NCH, body, 0)
    plsc.subcore_barrier()
    off = c * NPAD + s * RPT
    pltpu.sync_copy(hs_sh.at[pl.ds(s * RPT, RPT)], outs_hbm.at[pl.ds(off, RPT)])
    pltpu.sync_copy(hd_sh.at[pl.ds(s * RPT, RPT)], outd_hbm.at[pl.ds(off, RPT)])


@functools.partial(
    pl.kernel,
    mesh=_mesh,
    compiler_params=pltpu.CompilerParams(use_tc_tiling_on_sc=False),
    out_type=jax.ShapeDtypeStruct((NC * N, D), jnp.float32),
    scratch_types=[
        pltpu.VMEM((NCH, CH), jnp.int32),
        pltpu.VMEM((NCH, CH), jnp.int32),
        pltpu.VMEM((CH, D), jnp.float32),
        pltpu.VMEM((CH, D), jnp.float32),
        pltpu.VMEM((64, D), jnp.float32),
        pltpu.VMEM_SHARED((10000, D), jnp.float32),
        pltpu.SemaphoreType.DMA,
        pltpu.SemaphoreType.DMA,
        pltpu.SemaphoreType.DMA,
        pltpu.SemaphoreType.DMA,
    ],
)
def _agg_sc(g_hbm, src_hbm, dst_hbm, out_hbm,
            src_v, dst_v, rows0, rows1, zv, agg_sh, gs0, gs1, ss0, ss1):
    c = lax.axis_index("c")
    s = lax.axis_index("s")
    w = _worker_id()

    def zbody(i, carry):
        zv[i // 8, pl.ds((i % 8) * 16, 16)] = jnp.zeros((16,), jnp.float32)
        return carry

    lax.fori_loop(0, 64 * 8, zbody, 0)

    def zcopy(k, carry):
        pltpu.sync_copy(zv, agg_sh.at[pl.ds(s * 625 + k * 64, 64)])
        return carry

    lax.fori_loop(0, 9, zcopy, 0)
    pltpu.sync_copy(zv.at[pl.ds(0, 49)], agg_sh.at[pl.ds(s * 625 + 576, 49)])

    pltpu.sync_copy(src_hbm.at[w], src_v)
    pltpu.sync_copy(dst_hbm.at[w], dst_v)
    plsc.subcore_barrier()

    pltpu.async_copy(g_hbm.at[src_v.at[0]], rows0, gs0)
    pltpu.async_copy(g_hbm.at[src_v.at[1]], rows1, gs1)

    def body(i, carry):
        j0 = 2 * i
        j1 = j0 + 1
        pltpu.make_async_copy(g_hbm.at[src_v.at[j0]], rows0, gs0).wait()
        pltpu.async_copy(rows0, agg_sh.at[dst_v.at[j0]], ss0, add=True)

        @pl.when(j1 < NCH)
        def _():
            pltpu.make_async_copy(g_hbm.at[src_v.at[j1]], rows1, gs1).wait()
            pltpu.async_copy(rows1, agg_sh.at[dst_v.at[j1]], ss1, add=True)

        pltpu.make_async_copy(rows0, agg_sh.at[dst_v.at[j0]], ss0).wait()

        @pl.when(j0 + 2 < NCH)
        def _():
            pltpu.async_copy(g_hbm.at[src_v.at[j0 + 2]], rows0, gs0)

        @pl.when(j1 < NCH)
        def _():
            pltpu.make_async_copy(rows1, agg_sh.at[dst_v.at[j1]], ss1).wait()

            @pl.when(j1 + 2 < NCH)
            def _():
                pltpu.async_copy(g_hbm.at[src_v.at[j1 + 2]], rows1, gs1)

        return carry

    lax.fori_loop(0, (NCH + 1) // 2, body, 0)
    plsc.subcore_barrier()
    off = c * N + s * 625
    pltpu.sync_copy(agg_sh.at[pl.ds(s * 625, 625)], out_hbm.at[pl.ds(off, 625)])


def _norm_tc_body(feat_ref, deg_ref, g_ref):
    d = deg_ref[:N, 0:1] + deg_ref[:N, 1:2]
    norm = lax.rsqrt(jnp.maximum(d, 1.0))
    g_ref[...] = feat_ref[...] * norm


def _norm_tc(feat, deg_t):
    return pl.pallas_call(
        _norm_tc_body,
        out_shape=jax.ShapeDtypeStruct((N, D), jnp.float32),
    )(feat, deg_t)


def _tail_tc_body(aggp_ref, deg_ref, w_ref, b_ref, gamma_ref, beta_ref,
                  a1_ref, a2_ref, out_ref):
    agg = aggp_ref[:N, :] + aggp_ref[N:2 * N, :]
    d = deg_ref[:N, 0:1] + deg_ref[:N, 1:2]
    norm = lax.rsqrt(jnp.maximum(d, 1.0))
    h = agg * norm
    h = lax.dot_general(h, w_ref[...], (((1,), (0,)), ((), ())),
                        preferred_element_type=jnp.float32,
                        precision=lax.Precision.HIGHEST)
    h = h + b_ref[...]
    h = jnp.where(h >= 0, h, a1_ref[...] * h)
    mean = jnp.mean(h, axis=0, keepdims=True)
    var = jnp.mean((h - mean) ** 2, axis=0, keepdims=True)
    h = (h - mean) * lax.rsqrt(var + 1e-5) * gamma_ref[...] + beta_ref[...]
    out_ref[...] = jnp.where(h >= 0, h, a2_ref[...] * h)


def _tail_tc(aggp, deg_in_t, W, b, gamma, beta, a1, a2):
    return pl.pallas_call(
        _tail_tc_body,
        out_shape=jax.ShapeDtypeStruct((N, D), jnp.float32),
    )(aggp, deg_in_t, W, b.reshape(1, D), gamma.reshape(1, D),
      beta.reshape(1, D), a1.reshape(1, 1), a2.reshape(1, 1))


def kernel(feat, edge_index, W, b, a1, gamma, beta, a2):
    src = edge_index[0].astype(jnp.int32).reshape(NW, NCH, CH)
    dst = edge_index[1].astype(jnp.int32).reshape(NW, NCH, CH)
    zeros_h = jnp.zeros((NPAD,), jnp.float32)

    hs, hd = _hist_sc(src, dst, zeros_h)
    deg_out_t = hs.reshape(NC, NPAD).T
    deg_in_t = hd.reshape(NC, NPAD).T

    g = _norm_tc(feat, deg_out_t)
    aggp = _agg_sc(g, src, dst)
    return _tail_tc(aggp, deg_in_t, W, b, gamma, beta,
                    jnp.asarray(a1, jnp.float32), jnp.asarray(a2, jnp.float32))


# speedup vs baseline: 10.7234x; 1.0677x over previous
import functools

import jax
import jax.numpy as jnp
from jax import lax
from jax.experimental import pallas as pl
from jax.experimental.pallas import tpu as pltpu
from jax.experimental.pallas import tpu_sc as plsc

N = 10000
E = 320000
D = 128

NC = 2
NS = 16
NW = NC * NS

NPAD = 10240
RPT = NPAD // NS
EW = E // NW
CH = 80
NCH = EW // CH
CHH = 80
NCHH = EW // CHH

_mesh = plsc.VectorSubcoreMesh(core_axis_name="c", subcore_axis_name="s")


def _worker_id():
    return lax.axis_index("s") * NC + lax.axis_index("c")


@functools.partial(
    pl.kernel,
    mesh=_mesh,
    out_type=[
        jax.ShapeDtypeStruct((NC * NPAD,), jnp.float32),
        jax.ShapeDtypeStruct((NC * NPAD,), jnp.float32),
    ],
    scratch_types=[
        pltpu.VMEM((NCHH, CHH), jnp.int32),
        pltpu.VMEM((NCHH, CHH), jnp.int32),
        pltpu.VMEM((CHH,), jnp.float32),
        pltpu.VMEM_SHARED((NPAD,), jnp.float32),
        pltpu.VMEM_SHARED((NPAD,), jnp.float32),
        pltpu.SemaphoreType.DMA,
        pltpu.SemaphoreType.DMA,
    ],
)
def _hist_sc(src_hbm, dst_hbm, zeros_hbm, outs_hbm, outd_hbm,
             src_v, dst_v, ones_v, hs_sh, hd_sh, sa, sb):
    c = lax.axis_index("c")
    s = lax.axis_index("s")
    w = _worker_id()

    @pl.when(s == 0)
    def _():
        pltpu.sync_copy(zeros_hbm, hs_sh)
        pltpu.sync_copy(zeros_hbm, hd_sh)

    for j in range(CHH // 16):
        ones_v[pl.ds(j * 16, 16)] = jnp.ones((16,), jnp.float32)
    pltpu.sync_copy(src_hbm.at[w], src_v)
    pltpu.sync_copy(dst_hbm.at[w], dst_v)
    plsc.subcore_barrier()

    def body(j, carry):
        pltpu.async_copy(ones_v, hs_sh.at[src_v.at[j]], sa, add=True)
        pltpu.async_copy(ones_v, hd_sh.at[dst_v.at[j]], sb, add=True)
        return carry

    lax.fori_loop(0, NCHH, body, 0)

    def drain(j, carry):
        pltpu.make_async_copy(ones_v, hs_sh.at[src_v.at[0]], sa).wait()
        pltpu.make_async_copy(ones_v, hd_sh.at[dst_v.at[0]], sb).wait()
        return carry

    lax.fori_loop(0, NCHH, drain, 0)
    plsc.subcore_barrier()
    off = c * NPAD + s * RPT
    pltpu.sync_copy(hs_sh.at[pl.ds(s * RPT, RPT)], outs_hbm.at[pl.ds(off, RPT)])
    pltpu.sync_copy(hd_sh.at[pl.ds(s * RPT, RPT)], outd_hbm.at[pl.ds(off, RPT)])


@functools.partial(
    pl.kernel,
    mesh=_mesh,
    compiler_params=pltpu.CompilerParams(use_tc_tiling_on_sc=False),
    out_type=jax.ShapeDtypeStruct((NC * N, D), jnp.float32),
    scratch_types=[
        pltpu.VMEM((NCH, CH), jnp.int32),
        pltpu.VMEM((NCH, CH), jnp.int32),
        pltpu.VMEM((CH, D), jnp.float32),
        pltpu.VMEM((CH, D), jnp.float32),
        pltpu.VMEM((64, D), jnp.float32),
        pltpu.VMEM_SHARED((10000, D), jnp.float32),
        pltpu.SemaphoreType.DMA,
        pltpu.SemaphoreType.DMA,
        pltpu.SemaphoreType.DMA,
        pltpu.SemaphoreType.DMA,
    ],
)
def _agg_sc(g_hbm, src_hbm, dst_hbm, out_hbm,
            src_v, dst_v, rows0, rows1, zv, agg_sh, gs0, gs1, ss0, ss1):
    c = lax.axis_index("c")
    s = lax.axis_index("s")
    w = _worker_id()

    def zbody(i, carry):
        zv[i // 8, pl.ds((i % 8) * 16, 16)] = jnp.zeros((16,), jnp.float32)
        return carry

    lax.fori_loop(0, 64 * 8, zbody, 0)

    def zcopy(k, carry):
        pltpu.sync_copy(zv, agg_sh.at[pl.ds(s * 625 + k * 64, 64)])
        return carry

    lax.fori_loop(0, 9, zcopy, 0)
    pltpu.sync_copy(zv.at[pl.ds(0, 49)], agg_sh.at[pl.ds(s * 625 + 576, 49)])

    pltpu.sync_copy(src_hbm.at[w], src_v)
    pltpu.sync_copy(dst_hbm.at[w], dst_v)
    plsc.subcore_barrier()

    pltpu.async_copy(g_hbm.at[src_v.at[0]], rows0, gs0)
    pltpu.async_copy(g_hbm.at[src_v.at[1]], rows1, gs1)

    def body(i, carry):
        j0 = 2 * i
        j1 = j0 + 1
        pltpu.make_async_copy(g_hbm.at[src_v.at[j0]], rows0, gs0).wait()
        pltpu.async_copy(rows0, agg_sh.at[dst_v.at[j0]], ss0, add=True)

        @pl.when(j1 < NCH)
        def _():
            pltpu.make_async_copy(g_hbm.at[src_v.at[j1]], rows1, gs1).wait()
            pltpu.async_copy(rows1, agg_sh.at[dst_v.at[j1]], ss1, add=True)

        pltpu.make_async_copy(rows0, agg_sh.at[dst_v.at[j0]], ss0).wait()

        @pl.when(j0 + 2 < NCH)
        def _():
            pltpu.async_copy(g_hbm.at[src_v.at[j0 + 2]], rows0, gs0)

        @pl.when(j1 < NCH)
        def _():
            pltpu.make_async_copy(rows1, agg_sh.at[dst_v.at[j1]], ss1).wait()

            @pl.when(j1 + 2 < NCH)
            def _():
                pltpu.async_copy(g_hbm.at[src_v.at[j1 + 2]], rows1, gs1)

        return carry

    lax.fori_loop(0, (NCH + 1) // 2, body, 0)
    plsc.subcore_barrier()
    off = c * N + s * 625
    pltpu.sync_copy(agg_sh.at[pl.ds(s * 625, 625)], out_hbm.at[pl.ds(off, 625)])


def _norm_tc_body(feat_ref, deg_ref, g_ref):
    d = deg_ref[:N, 0:1] + deg_ref[:N, 1:2]
    norm = lax.rsqrt(jnp.maximum(d, 1.0))
    g_ref[...] = feat_ref[...] * norm


def _norm_tc(feat, deg_t):
    return pl.pallas_call(
        _norm_tc_body,
        out_shape=jax.ShapeDtypeStruct((N, D), jnp.float32),
    )(feat, deg_t)


def _tail_tc_body(aggp_ref, deg_ref, w_ref, b_ref, gamma_ref, beta_ref,
                  a1_ref, a2_ref, out_ref):
    agg = aggp_ref[:N, :] + aggp_ref[N:2 * N, :]
    d = deg_ref[:N, 0:1] + deg_ref[:N, 1:2]
    norm = lax.rsqrt(jnp.maximum(d, 1.0))
    h = agg * norm
    h = lax.dot_general(h, w_ref[...], (((1,), (0,)), ((), ())),
                        preferred_element_type=jnp.float32,
                        precision=lax.Precision.HIGHEST)
    h = h + b_ref[...]
    h = jnp.where(h >= 0, h, a1_ref[...] * h)
    mean = jnp.mean(h, axis=0, keepdims=True)
    var = jnp.mean((h - mean) ** 2, axis=0, keepdims=True)
    h = (h - mean) * lax.rsqrt(var + 1e-5) * gamma_ref[...] + beta_ref[...]
    out_ref[...] = jnp.where(h >= 0, h, a2_ref[...] * h)


def _tail_tc(aggp, deg_in_t, W, b, gamma, beta, a1, a2):
    return pl.pallas_call(
        _tail_tc_body,
        out_shape=jax.ShapeDtypeStruct((N, D), jnp.float32),
    )(aggp, deg_in_t, W, b.reshape(1, D), gamma.reshape(1, D),
      beta.reshape(1, D), a1.reshape(1, 1), a2.reshape(1, 1))


def kernel(feat, edge_index, W, b, a1, gamma, beta, a2):
    src = edge_index[0].astype(jnp.int32)
    dst = edge_index[1].astype(jnp.int32)
    src_h = src.reshape(NW, NCHH, CHH)
    dst_h = dst.reshape(NW, NCHH, CHH)
    src_a = src.reshape(NW, NCH, CH)
    dst_a = dst.reshape(NW, NCH, CH)
    zeros_h = jnp.zeros((NPAD,), jnp.float32)

    hs, hd = _hist_sc(src_h, dst_h, zeros_h)
    deg_out_t = hs.reshape(NC, NPAD).T
    deg_in_t = hd.reshape(NC, NPAD).T

    g = _norm_tc(feat, deg_out_t)
    aggp = _agg_sc(g, src_a, dst_a)
    return _tail_tc(aggp, deg_in_t, W, b, gamma, beta,
                    jnp.asarray(a1, jnp.float32), jnp.asarray(a2, jnp.float32))


# speedup vs baseline: 10.9093x; 1.0173x over previous
import functools

import jax
import jax.numpy as jnp
from jax import lax
from jax.experimental import pallas as pl
from jax.experimental.pallas import tpu as pltpu
from jax.experimental.pallas import tpu_sc as plsc

N = 10000
E = 320000
D = 128

NC = 2
NS = 16
NW = NC * NS

NPAD = 10240
RPT = NPAD // NS
EW = E // NW
CH = 80
NCH = EW // CH
CHH = 80
NCHH = EW // CHH

_mesh = plsc.VectorSubcoreMesh(core_axis_name="c", subcore_axis_name="s")


def _worker_id():
    return lax.axis_index("s") * NC + lax.axis_index("c")


@functools.partial(
    pl.kernel,
    mesh=_mesh,
    compiler_params=pltpu.CompilerParams(use_tc_tiling_on_sc=False),
    out_type=[
        jax.ShapeDtypeStruct((NC * NPAD,), jnp.float32),
        jax.ShapeDtypeStruct((NC * NPAD,), jnp.float32),
    ],
    scratch_types=[
        pltpu.VMEM((NCHH, CHH), jnp.int32),
        pltpu.VMEM((NCHH, CHH), jnp.int32),
        pltpu.VMEM((CHH,), jnp.float32),
        pltpu.VMEM_SHARED((NPAD,), jnp.float32),
        pltpu.VMEM_SHARED((NPAD,), jnp.float32),
        pltpu.SemaphoreType.DMA,
        pltpu.SemaphoreType.DMA,
    ],
)
def _hist_sc(src_hbm, dst_hbm, zeros_hbm, outs_hbm, outd_hbm,
             src_v, dst_v, ones_v, hs_sh, hd_sh, sa, sb):
    c = lax.axis_index("c")
    s = lax.axis_index("s")
    w = _worker_id()

    @pl.when(s == 0)
    def _():
        pltpu.sync_copy(zeros_hbm, hs_sh)
        pltpu.sync_copy(zeros_hbm, hd_sh)

    for j in range(CHH // 16):
        ones_v[pl.ds(j * 16, 16)] = jnp.ones((16,), jnp.float32)
    pltpu.sync_copy(src_hbm.at[w], src_v)
    pltpu.sync_copy(dst_hbm.at[w], dst_v)
    plsc.subcore_barrier()

    def body(j, carry):
        pltpu.async_copy(ones_v, hs_sh.at[src_v.at[j]], sa, add=True)
        pltpu.async_copy(ones_v, hd_sh.at[dst_v.at[j]], sb, add=True)
        return carry

    lax.fori_loop(0, NCHH, body, 0)

    def drain(j, carry):
        pltpu.make_async_copy(ones_v, hs_sh.at[src_v.at[0]], sa).wait()
        pltpu.make_async_copy(ones_v, hd_sh.at[dst_v.at[0]], sb).wait()
        return carry

    lax.fori_loop(0, NCHH, drain, 0)
    plsc.subcore_barrier()
    off = c * NPAD + s * RPT
    pltpu.sync_copy(hs_sh.at[pl.ds(s * RPT, RPT)], outs_hbm.at[pl.ds(off, RPT)])
    pltpu.sync_copy(hd_sh.at[pl.ds(s * RPT, RPT)], outd_hbm.at[pl.ds(off, RPT)])


@functools.partial(
    pl.kernel,
    mesh=_mesh,
    compiler_params=pltpu.CompilerParams(use_tc_tiling_on_sc=False),
    out_type=jax.ShapeDtypeStruct((NC * N, D), jnp.float32),
    scratch_types=[
        pltpu.VMEM((NCH, CH), jnp.int32),
        pltpu.VMEM((NCH, CH), jnp.int32),
        pltpu.VMEM((CH, D), jnp.float32),
        pltpu.VMEM((CH, D), jnp.float32),
        pltpu.VMEM((64, D), jnp.float32),
        pltpu.VMEM_SHARED((10000, D), jnp.float32),
        pltpu.SemaphoreType.DMA,
        pltpu.SemaphoreType.DMA,
        pltpu.SemaphoreType.DMA,
        pltpu.SemaphoreType.DMA,
    ],
)
def _agg_sc(g_hbm, src_hbm, dst_hbm, out_hbm,
            src_v, dst_v, rows0, rows1, zv, agg_sh, gs0, gs1, ss0, ss1):
    c = lax.axis_index("c")
    s = lax.axis_index("s")
    w = _worker_id()

    def zbody(i, carry):
        zv[i // 8, pl.ds((i % 8) * 16, 16)] = jnp.zeros((16,), jnp.float32)
        return carry

    lax.fori_loop(0, 64 * 8, zbody, 0)

    def zcopy(k, carry):
        pltpu.sync_copy(zv, agg_sh.at[pl.ds(s * 625 + k * 64, 64)])
        return carry

    lax.fori_loop(0, 9, zcopy, 0)
    pltpu.sync_copy(zv.at[pl.ds(0, 49)], agg_sh.at[pl.ds(s * 625 + 576, 49)])

    pltpu.sync_copy(src_hbm.at[w], src_v)
    pltpu.sync_copy(dst_hbm.at[w], dst_v)
    plsc.subcore_barrier()

    pltpu.async_copy(g_hbm.at[src_v.at[0]], rows0, gs0)
    pltpu.async_copy(g_hbm.at[src_v.at[1]], rows1, gs1)

    def body(i, carry):
        j0 = 2 * i
        j1 = j0 + 1
        pltpu.make_async_copy(g_hbm.at[src_v.at[j0]], rows0, gs0).wait()
        pltpu.async_copy(rows0, agg_sh.at[dst_v.at[j0]], ss0, add=True)

        @pl.when(j1 < NCH)
        def _():
            pltpu.make_async_copy(g_hbm.at[src_v.at[j1]], rows1, gs1).wait()
            pltpu.async_copy(rows1, agg_sh.at[dst_v.at[j1]], ss1, add=True)

        pltpu.make_async_copy(rows0, agg_sh.at[dst_v.at[j0]], ss0).wait()

        @pl.when(j0 + 2 < NCH)
        def _():
            pltpu.async_copy(g_hbm.at[src_v.at[j0 + 2]], rows0, gs0)

        @pl.when(j1 < NCH)
        def _():
            pltpu.make_async_copy(rows1, agg_sh.at[dst_v.at[j1]], ss1).wait()

            @pl.when(j1 + 2 < NCH)
            def _():
                pltpu.async_copy(g_hbm.at[src_v.at[j1 + 2]], rows1, gs1)

        return carry

    lax.fori_loop(0, (NCH + 1) // 2, body, 0)
    plsc.subcore_barrier()
    off = c * N + s * 625
    pltpu.sync_copy(agg_sh.at[pl.ds(s * 625, 625)], out_hbm.at[pl.ds(off, 625)])


def _norm_tc_body(feat_ref, deg_ref, g_ref):
    d = deg_ref[:N, 0:1] + deg_ref[:N, 1:2]
    norm = lax.rsqrt(jnp.maximum(d, 1.0))
    g_ref[...] = feat_ref[...] * norm


def _norm_tc(feat, deg_t):
    return pl.pallas_call(
        _norm_tc_body,
        out_shape=jax.ShapeDtypeStruct((N, D), jnp.float32),
    )(feat, deg_t)


def _tail_tc_body(aggp_ref, deg_ref, w_ref, b_ref, gamma_ref, beta_ref,
                  a1_ref, a2_ref, out_ref):
    agg = aggp_ref[:N, :] + aggp_ref[N:2 * N, :]
    d = deg_ref[:N, 0:1] + deg_ref[:N, 1:2]
    norm = lax.rsqrt(jnp.maximum(d, 1.0))
    h = agg * norm
    h = lax.dot_general(h, w_ref[...], (((1,), (0,)), ((), ())),
                        preferred_element_type=jnp.float32,
                        precision=lax.Precision.HIGHEST)
    h = h + b_ref[...]
    h = jnp.where(h >= 0, h, a1_ref[...] * h)
    mean = jnp.mean(h, axis=0, keepdims=True)
    var = jnp.mean((h - mean) ** 2, axis=0, keepdims=True)
    h = (h - mean) * lax.rsqrt(var + 1e-5) * gamma_ref[...] + beta_ref[...]
    out_ref[...] = jnp.where(h >= 0, h, a2_ref[...] * h)


def _tail_tc(aggp, deg_in_t, W, b, gamma, beta, a1, a2):
    return pl.pallas_call(
        _tail_tc_body,
        out_shape=jax.ShapeDtypeStruct((N, D), jnp.float32),
    )(aggp, deg_in_t, W, b.reshape(1, D), gamma.reshape(1, D),
      beta.reshape(1, D), a1.reshape(1, 1), a2.reshape(1, 1))


def kernel(feat, edge_index, W, b, a1, gamma, beta, a2):
    src = edge_index[0].astype(jnp.int32)
    dst = edge_index[1].astype(jnp.int32)
    src_h = src.reshape(NW, NCHH, CHH)
    dst_h = dst.reshape(NW, NCHH, CHH)
    src_a = src.reshape(NW, NCH, CH)
    dst_a = dst.reshape(NW, NCH, CH)
    zeros_h = jnp.zeros((NPAD,), jnp.float32)

    hs, hd = _hist_sc(src_h, dst_h, zeros_h)
    deg_out_t = hs.reshape(NC, NPAD).T
    deg_in_t = hd.reshape(NC, NPAD).T

    g = _norm_tc(feat, deg_out_t)
    aggp = _agg_sc(g, src_a, dst_a)
    return _tail_tc(aggp, deg_in_t, W, b, gamma, beta,
                    jnp.asarray(a1, jnp.float32), jnp.asarray(a2, jnp.float32))
